# baseline (speedup 1.0000x reference)
import functools

import jax
import jax.numpy as jnp
from jax import lax
from jax.experimental import pallas as pl
from jax.experimental.pallas import tpu as pltpu
from jax.experimental.pallas import tpu_sc as plsc

B, O, I, D = 32, 16, 16, 1024
K = 1025
SEG = 512
OG = 4
BG = 4
NO_GROUPS = O // OG
ROWS = OG * I
TABLE_WORDS = ROWS * K
COEF_WORDS = ROWS * SEG
XSLAB = I * D
ACC_WORDS = OG * D
ROWCHUNK = 8
CHUNK_WORDS = ROWCHUNK * K
NCHUNK = ROWS // ROWCHUNK
UNROLL = 2
NV = D // (16 * UNROLL)


def _body(w_hbm, x_hbm, out_hbm, ta, tb, tc, xb_v, acc_v, rowbuf, sem_a, sem_b):
    wid = lax.axis_index("s") * 2 + lax.axis_index("c")
    o_group = wid % NO_GROUPS
    b_group = wid // NO_GROUPS
    w_base = o_group * TABLE_WORDS

    lane2 = 2 * lax.iota(jnp.int32, 16)

    bufs = [rowbuf, xb_v.at[pl.ds(0, CHUNK_WORDS)]]
    sems = [sem_a, sem_b]
    copies = [None, None]
    copies[0] = pltpu.async_copy(
        w_hbm.at[pl.ds(w_base, CHUNK_WORDS)], bufs[0], sems[0]
    )
    for c in range(NCHUNK):
        cur = c % 2
        if c + 1 < NCHUNK:
            copies[1 - cur] = pltpu.async_copy(
                w_hbm.at[pl.ds(w_base + (c + 1) * CHUNK_WORDS, CHUNK_WORDS)],
                bufs[1 - cur], sems[1 - cur],
            )
        copies[cur].wait()
        buf = bufs[cur]

        def build_row(lr, _, buf=buf, c=c):
            r = c * ROWCHUNK + lr
            for g in range(SEG // 16):
                qi = lr * K + 32 * g + lane2
                w0 = plsc.load_gather(buf, [qi])
                w1 = plsc.load_gather(buf, [qi + 1])
                w2 = plsc.load_gather(buf, [qi + 2])
                off = r * SEG + g * 16
                ta[pl.ds(off, 16)] = w1
                tb[pl.ds(off, 16)] = 0.5 * (w2 - w0)
                tc[pl.ds(off, 16)] = 0.5 * (w0 + w2) - w1
            return ()

        lax.fori_loop(0, ROWCHUNK, build_row, ())

    for b in range(BG):
        b_abs = b_group * BG + b
        pltpu.sync_copy(x_hbm.at[pl.ds(b_abs * XSLAB, XSLAB)], xb_v)

        def dloop(v, _):
            dvs = [v * (16 * UNROLL) + u * 16 for u in range(UNROLL)]
            accs = [[jnp.zeros((16,), jnp.float32) for _ in range(OG)]
                    for _ in range(UNROLL)]
            for i in range(I):
                for u in range(UNROLL):
                    xv = xb_v[pl.ds(i * D + dvs[u], 16)]
                    seg = ((xv + 1.0) * 256.0).astype(jnp.int32)
                    seg = jnp.minimum(jnp.maximum(seg, 0), SEG - 1)
                    x_min = seg.astype(jnp.float32) * (1.0 / 256.0) - 1.0
                    t = (xv - x_min) * 512.0 - 1.0
                    t2 = t * t
                    for o in range(OG):
                        q = seg + (o * I + i) * SEG
                        av = plsc.load_gather(ta, [q])
                        bv = plsc.load_gather(tb, [q])
                        cv = plsc.load_gather(tc, [q])
                        accs[u][o] = accs[u][o] + (av + t * bv + t2 * cv)
            for u in range(UNROLL):
                for o in range(OG):
                    acc_v[pl.ds(o * D + dvs[u], 16)] = accs[u][o]
            return ()

        lax.fori_loop(0, NV, dloop, ())
        out_off = (b_abs * O + o_group * OG) * D
        pltpu.sync_copy(acc_v, out_hbm.at[pl.ds(out_off, ACC_WORDS)])


@jax.jit
def _piecewise_sc(x_flat, w_flat):
    mesh = plsc.VectorSubcoreMesh(core_axis_name="c", subcore_axis_name="s")
    kfn = functools.partial(
        pl.kernel,
        mesh=mesh,
        out_type=jax.ShapeDtypeStruct((B * O * D,), jnp.float32),
        scratch_types=[
            pltpu.VMEM((COEF_WORDS,), jnp.float32),
            pltpu.VMEM((COEF_WORDS,), jnp.float32),
            pltpu.VMEM((COEF_WORDS,), jnp.float32),
            pltpu.VMEM((XSLAB,), jnp.float32),
            pltpu.VMEM((ACC_WORDS,), jnp.float32),
            pltpu.VMEM((CHUNK_WORDS,), jnp.float32),
            pltpu.SemaphoreType.DMA,
            pltpu.SemaphoreType.DMA,
        ],
        compiler_params=pltpu.CompilerParams(needs_layout_passes=False),
    )(_body)
    return kfn(w_flat, x_flat)


def kernel(x, w):
    x_flat = x.reshape(B * I * D)
    w_flat = w.reshape(O * I * K)
    out = _piecewise_sc(x_flat, w_flat)
    return out.reshape(B, O, D)


# speedup vs baseline: 1.2255x; 1.2255x over previous
import functools

import jax
import jax.numpy as jnp
from jax import lax
from jax.experimental import pallas as pl
from jax.experimental.pallas import tpu as pltpu
from jax.experimental.pallas import tpu_sc as plsc

B, O, I, D = 32, 16, 16, 1024
K = 1025
SEGMENTS = 512
OG = 4
BG = 4
NO_GROUPS = O // OG
TABLE_WORDS = OG * I * K
XSLAB = I * D
ACC_WORDS = OG * D
UNROLL = 2
NV = D // (16 * UNROLL)


def _interp_step(table_v, xb_v, i, dv):
    xv = xb_v[pl.ds(i * D + dv, 16)]
    seg = ((xv + 1.0) * 256.0).astype(jnp.int32)
    seg = jnp.minimum(jnp.maximum(seg, 0), SEGMENTS - 1)
    x_min = seg.astype(jnp.float32) * (1.0 / 256.0) - 1.0
    t = (xv - x_min) * 512.0 - 1.0
    t2 = t * t
    c0 = 0.5 * (t2 - t)
    c1 = 1.0 - t2
    c2 = 0.5 * (t2 + t)
    base = 2 * seg
    out = []
    for o in range(OG):
        ro = (o * I + i) * K
        g0 = plsc.load_gather(table_v, [base + ro])
        g1 = plsc.load_gather(table_v, [base + (ro + 1)])
        g2 = plsc.load_gather(table_v, [base + (ro + 2)])
        out.append(c0 * g0 + c1 * g1 + c2 * g2)
    return out


def _body(w_hbm, x_hbm, out_hbm, table_v, xb0_v, xb1_v, acc0_v, acc1_v,
          sem, out_sem0, out_sem1):
    wid = lax.axis_index("s") * 2 + lax.axis_index("c")
    o_group = wid % NO_GROUPS
    b_group = wid // NO_GROUPS

    tbl_copy = pltpu.async_copy(
        w_hbm.at[pl.ds(o_group * TABLE_WORDS, TABLE_WORDS)], table_v, sem
    )
    b0_abs = b_group * BG
    x_copy = pltpu.async_copy(
        x_hbm.at[pl.ds(b0_abs * XSLAB, XSLAB)], xb0_v, sem
    )
    tbl_copy.wait()
    x_copy.wait()

    xbufs = [xb0_v, xb1_v]
    accbufs = [acc0_v, acc1_v]
    out_sems = [out_sem0, out_sem1]
    out_copies = [None, None]
    for b in range(BG):
        b_abs = b_group * BG + b
        xbuf = xbufs[b % 2]
        accbuf = accbufs[b % 2]
        if b + 1 < BG:
            nxt = pltpu.async_copy(
                x_hbm.at[pl.ds((b_abs + 1) * XSLAB, XSLAB)], xbufs[1 - b % 2], sem
            )
        if out_copies[b % 2] is not None:
            out_copies[b % 2].wait()

        def dloop(v, _, xbuf=xbuf, accbuf=accbuf):
            dvs = [v * (16 * UNROLL) + u * 16 for u in range(UNROLL)]
            accs = [[jnp.zeros((16,), jnp.float32) for _ in range(OG)]
                    for _ in range(UNROLL)]
            for i in range(I):
                for u in range(UNROLL):
                    contrib = _interp_step(table_v, xbuf, i, dvs[u])
                    for o in range(OG):
                        accs[u][o] = accs[u][o] + contrib[o]
            for u in range(UNROLL):
                for o in range(OG):
                    accbuf[pl.ds(o * D + dvs[u], 16)] = accs[u][o]
            return ()

        lax.fori_loop(0, NV, dloop, ())
        out_off = (b_abs * O + o_group * OG) * D
        out_copies[b % 2] = pltpu.async_copy(
            accbuf, out_hbm.at[pl.ds(out_off, ACC_WORDS)], out_sems[b % 2]
        )
        if b + 1 < BG:
            nxt.wait()
    for cp in out_copies:
        if cp is not None:
            cp.wait()


@jax.jit
def _piecewise_sc(x_flat, w_flat):
    mesh = plsc.VectorSubcoreMesh(core_axis_name="c", subcore_axis_name="s")
    kfn = functools.partial(
        pl.kernel,
        mesh=mesh,
        out_type=jax.ShapeDtypeStruct((B * O * D,), jnp.float32),
        scratch_types=[
            pltpu.VMEM((TABLE_WORDS,), jnp.float32),
            pltpu.VMEM((XSLAB,), jnp.float32),
            pltpu.VMEM((XSLAB,), jnp.float32),
            pltpu.VMEM((ACC_WORDS,), jnp.float32),
            pltpu.VMEM((ACC_WORDS,), jnp.float32),
            pltpu.SemaphoreType.DMA,
            pltpu.SemaphoreType.DMA,
            pltpu.SemaphoreType.DMA,
        ],
        compiler_params=pltpu.CompilerParams(needs_layout_passes=False),
    )(_body)
    return kfn(w_flat, x_flat)


def kernel(x, w):
    x_flat = x.reshape(B * I * D)
    w_flat = w.reshape(O * I * K)
    out = _piecewise_sc(x_flat, w_flat)
    return out.reshape(B, O, D)
